# baseline (speedup 1.0000x reference)
import functools

import jax
import jax.numpy as jnp
import numpy as np
from jax import lax
from jax.experimental import pallas as pl
from jax.experimental.pallas import tpu as pltpu
from jax.experimental.pallas import tpu_sc as plsc

B, S, D = 1, 2048, 2048
H, KVH, HD = 16, 4, 128
E, K, F = 8, 2, 768
EPS = 1e-06
THETA = 10000.0
HALF = HD // 2

TS = 256
NQKV = (H + 2 * KVH) * HD
TQ = 512
SCALE = 1.0 / float(np.sqrt(HD))


def _rms(x):
    return jax.lax.rsqrt(jnp.mean(x * x, axis=-1, keepdims=True) + EPS)


NHT = H + 2 * KVH


def _qkv_body(x_ref, ln1_ref, w_ref, wh_ref, o_ref):
    i = pl.program_id(0)
    x = x_ref[...]
    h = x * _rms(x) * ln1_ref[...]
    y = jnp.dot(h, w_ref[...], preferred_element_type=jnp.float32)
    y3 = y.reshape(TS, NHT, HD)
    yn = y3 * _rms(y3) * wh_ref[...][None]
    pos = (i * TS + lax.broadcasted_iota(jnp.int32, (TS, HALF), 0)
           ).astype(jnp.float32)
    inv = jnp.exp(lax.broadcasted_iota(jnp.int32, (TS, HALF), 1)
                  .astype(jnp.float32) * (-np.log(THETA) / HALF))
    f = pos * inv
    cos = jnp.cos(f)[:, None, :]
    sin = jnp.sin(f)[:, None, :]
    x1 = yn[..., :HALF]
    x2 = yn[..., HALF:]
    rot = jnp.concatenate([x1 * cos - x2 * sin, x2 * cos + x1 * sin], axis=-1)
    hiota = lax.broadcasted_iota(jnp.int32, (TS, NHT, HD), 1)
    o_ref[...] = jnp.where(hiota >= H + KVH, y3, rot).reshape(TS, NQKV)


def _qkv(x, ln1_w, w_all, qn, kn):
    wh = jnp.concatenate([
        jnp.broadcast_to(qn, (H, HD)),
        jnp.broadcast_to(kn, (KVH, HD)),
        jnp.ones((KVH, HD), jnp.float32),
    ], axis=0)
    return pl.pallas_call(
        _qkv_body,
        grid=(S // TS,),
        in_specs=[
            pl.BlockSpec((TS, D), lambda i: (i, 0)),
            pl.BlockSpec((1, D), lambda i: (0, 0)),
            pl.BlockSpec((D, NQKV), lambda i: (0, 0)),
            pl.BlockSpec((NHT, HD), lambda i: (0, 0)),
        ],
        out_specs=pl.BlockSpec((TS, NQKV), lambda i: (i, 0)),
        out_shape=jax.ShapeDtypeStruct((S, NQKV), jnp.float32),
    )(x, ln1_w.reshape(1, D), w_all, wh)


LOG2E = float(np.log2(np.e))


def _attn_body(ii_ref, kk_ref, q_ref, k_ref, v_ref, o_ref, m_s, l_s, a_s):
    t = pl.program_id(1)
    i = ii_ref[t]
    kt = kk_ref[t]

    @pl.when(kt == 0)
    def _():
        m_s[...] = jnp.full((TQ, 2), -1e30, jnp.float32)
        l_s[...] = jnp.zeros((TQ, 2), jnp.float32)
        a_s[...] = jnp.zeros((TQ, 2 * HD), jnp.float32)

    kb = k_ref[pl.ds(kt * TQ, TQ), :]
    vb = v_ref[pl.ds(kt * TQ, TQ), :]
    riota = lax.broadcasted_iota(jnp.int32, (TQ, TQ), 0)
    ciota = lax.broadcasted_iota(jnp.int32, (TQ, TQ), 1)
    keep = (kt < i) | (riota >= ciota)
    for a in (0, 1):
        sl = pl.ds(a * HD, HD)
        qa = q_ref[:, sl] * (SCALE * LOG2E)
        s = lax.dot_general(qa, kb, (((1,), (1,)), ((), ())),
                            preferred_element_type=jnp.float32)
        s = jnp.where(keep, s, -1e30)
        m_prev = m_s[:, a:a + 1]
        m_new = jnp.maximum(m_prev, jnp.max(s, axis=1, keepdims=True))
        alpha = jnp.exp2(m_prev - m_new)
        p = jnp.exp2(s - m_new)
        l_new = l_s[:, a:a + 1] * alpha + jnp.sum(p, axis=1, keepdims=True)
        a_new = (a_s[:, sl] * alpha
                 + lax.dot_general(p, vb, (((1,), (0,)), ((), ())),
                                   preferred_element_type=jnp.float32))
        m_s[:, a:a + 1] = m_new
        l_s[:, a:a + 1] = l_new
        a_s[:, sl] = a_new

    @pl.when(kt == i)
    def _():
        for a in (0, 1):
            sl = pl.ds(a * HD, HD)
            o_ref[:, sl] = a_s[:, sl] / l_s[:, a:a + 1]


def _attn(qkv):
    nq = S // TQ
    rep = H // KVH
    tri = [(i, k) for i in range(nq) for k in range(i + 1)]
    iidx = jnp.asarray(np.array([x[0] for x in tri], np.int32))
    kidx = jnp.asarray(np.array([x[1] for x in tri], np.int32))
    return pl.pallas_call(
        _attn_body,
        grid_spec=pltpu.PrefetchScalarGridSpec(
            num_scalar_prefetch=2,
            grid=(H // 2, len(tri)),
            in_specs=[
                pl.BlockSpec((TQ, 2 * HD), lambda h2, t, ii, kk: (ii[t], h2)),
                pl.BlockSpec((S, HD),
                             lambda h2, t, ii, kk: (0, H + (2 * h2) // rep)),
                pl.BlockSpec((S, HD),
                             lambda h2, t, ii, kk: (0, H + KVH + (2 * h2) // rep)),
            ],
            out_specs=pl.BlockSpec((TQ, 2 * HD),
                                   lambda h2, t, ii, kk: (ii[t], h2)),
            scratch_shapes=[
                pltpu.VMEM((TQ, 2), jnp.float32),
                pltpu.VMEM((TQ, 2), jnp.float32),
                pltpu.VMEM((TQ, 2 * HD), jnp.float32),
            ],
        ),
        out_shape=jax.ShapeDtypeStruct((S, H * HD), jnp.float32),
        compiler_params=pltpu.CompilerParams(
            dimension_semantics=("parallel", "arbitrary")),
    )(iidx, kidx, qkv, qkv, qkv)


TNO = 512


def _wo_body(o_ref, w_ref, r_ref, y_ref):
    y_ref[...] = r_ref[...] + jnp.dot(o_ref[...], w_ref[...],
                                      preferred_element_type=jnp.float32)


def _wo(o, wo, resid):
    return pl.pallas_call(
        _wo_body,
        grid=(S // TS, D // TNO),
        in_specs=[
            pl.BlockSpec((TS, H * HD), lambda i, j: (i, 0)),
            pl.BlockSpec((H * HD, TNO), lambda i, j: (0, j)),
            pl.BlockSpec((TS, TNO), lambda i, j: (i, j)),
        ],
        out_specs=pl.BlockSpec((TS, TNO), lambda i, j: (i, j)),
        out_shape=jax.ShapeDtypeStruct((S, D), jnp.float32),
    )(o, wo, resid)


def _ln2_body(x_ref, w_ref, rw_ref, h_ref, lg_ref):
    x = x_ref[...]
    hh = x * _rms(x) * w_ref[...]
    h_ref[...] = hh.astype(jnp.bfloat16)
    lg_ref[...] = jnp.dot(hh, rw_ref[...], preferred_element_type=jnp.float32)


def _ln2(x, ln2_w, router_W):
    return pl.pallas_call(
        _ln2_body,
        grid=(S // TS,),
        in_specs=[
            pl.BlockSpec((TS, D), lambda i: (i, 0)),
            pl.BlockSpec((1, D), lambda i: (0, 0)),
            pl.BlockSpec((D, E), lambda i: (0, 0)),
        ],
        out_specs=[
            pl.BlockSpec((TS, D), lambda i: (i, 0)),
            pl.BlockSpec((TS, E), lambda i: (i, 0)),
        ],
        out_shape=[
            jax.ShapeDtypeStruct((S, D), jnp.bfloat16),
            jax.ShapeDtypeStruct((S, E), jnp.float32),
        ],
    )(x, ln2_w.reshape(1, D), router_W)


TM = 256
NT = (K * S) // TM + E
NPAD = NT * TM
CCH = 128
NCH = (K * S) // CCH


def _route_body(lg_ref, pos_ref, w_ref, texp_ref):
    lg = lg_ref[...]
    m = jnp.max(lg, axis=1, keepdims=True)
    p = jnp.exp(lg - m)
    p = p / jnp.sum(p, axis=1, keepdims=True)
    iota = lax.broadcasted_iota(jnp.int32, (S, E), 1)
    m1 = jnp.max(p, axis=1, keepdims=True)
    i1 = jnp.min(jnp.where(p == m1, iota, E), axis=1, keepdims=True)
    p2 = jnp.where(iota == i1, -1.0, p)
    m2 = jnp.max(p2, axis=1, keepdims=True)
    i2 = jnp.min(jnp.where(p2 == m2, iota, E), axis=1, keepdims=True)
    denom = m1 + m2
    w_ref[...] = jnp.concatenate([m1 / denom, m2 / denom], axis=0)
    idx_all = jnp.concatenate([i1, i2], axis=0)
    M = (lax.broadcasted_iota(jnp.int32, (K * S, E), 1) == idx_all
         ).astype(jnp.float32)
    M3 = M.reshape(NCH, CCH, E)
    tri = (lax.broadcasted_iota(jnp.int32, (CCH, CCH), 1)
           < lax.broadcasted_iota(jnp.int32, (CCH, CCH), 0)).astype(jnp.float32)
    trib = jnp.broadcast_to(tri, (NCH, CCH, CCH))
    pre = lax.dot_general(trib, M3, (((2,), (1,)), ((0,), (0,))),
                          preferred_element_type=jnp.float32)
    tot = jnp.sum(M3, axis=1)
    tri2 = (lax.broadcasted_iota(jnp.int32, (NCH, NCH), 1)
            < lax.broadcasted_iota(jnp.int32, (NCH, NCH), 0)).astype(jnp.float32)
    coff = jnp.dot(tri2, tot, preferred_element_type=jnp.float32)
    rank = (pre + coff[:, None, :]).reshape(K * S, E)
    counts = jnp.sum(M, axis=0, keepdims=True)
    pc = jnp.floor((counts + (TM - 1)) * (1.0 / TM)) * TM
    triu = (lax.broadcasted_iota(jnp.int32, (E, E), 0)
            < lax.broadcasted_iota(jnp.int32, (E, E), 1)).astype(jnp.float32)
    pad_off = jnp.dot(pc, triu, preferred_element_type=jnp.float32)
    posf = jnp.sum((rank + pad_off) * M, axis=1, keepdims=True)
    pos_ref[...] = posf.astype(jnp.int32)
    pad_end = pad_off + pc
    jtf = (lax.broadcasted_iota(jnp.int32, (NT, E), 0) * TM).astype(jnp.float32)
    texp = jnp.sum((pad_end <= jtf).astype(jnp.int32), axis=1, keepdims=True)
    texp_ref[...] = jnp.minimum(texp, E - 1)


def _route(logits):
    return pl.pallas_call(
        _route_body,
        out_shape=[
            jax.ShapeDtypeStruct((K * S, 1), jnp.int32),
            jax.ShapeDtypeStruct((K * S, 1), jnp.float32),
            jax.ShapeDtypeStruct((NT, 1), jnp.int32),
        ],
    )(logits)


SCH = 256


def _scat_body(pos_ref, w_ref, stok_ref, sw_ref):
    jcols = pl.program_id(0) * SCH + lax.broadcasted_iota(
        jnp.int32, (K * S, SCH), 1)
    cmp = pos_ref[...] == jcols
    it = lax.broadcasted_iota(jnp.int32, (K * S, 1), 0)
    tok = jnp.where(it >= S, it - S, it)
    stok = jnp.sum(jnp.where(cmp, tok, 0), axis=0, keepdims=True)
    sw = jnp.sum(jnp.where(cmp, w_ref[...], 0.0), axis=0, keepdims=True)
    stok_ref[...] = stok.reshape(1, 1, SCH)
    sw_ref[...] = sw.reshape(1, 1, SCH)


def _scat(pos, w):
    return pl.pallas_call(
        _scat_body,
        grid=(NPAD // SCH,),
        in_specs=[
            pl.BlockSpec((K * S, 1), lambda j: (0, 0)),
            pl.BlockSpec((K * S, 1), lambda j: (0, 0)),
        ],
        out_specs=[
            pl.BlockSpec((1, 1, SCH), lambda j: (j, 0, 0)),
            pl.BlockSpec((1, 1, SCH), lambda j: (j, 0, 0)),
        ],
        out_shape=[
            jax.ShapeDtypeStruct((NPAD // SCH, 1, SCH), jnp.int32),
            jax.ShapeDtypeStruct((NPAD // SCH, 1, SCH), jnp.float32),
        ],
    )(pos, w)


SC_NC, SC_NS = 2, 16
NW = SC_NC * SC_NS
DROWS = NPAD // NW
DCH = 24


def _disp_body(h_hbm, tok_hbm, out_hbm, idx_v, rows_v,
               gsem0, gsem1, ssem0, ssem1):
    wid = lax.axis_index("s") * SC_NC + lax.axis_index("c")
    base = wid * DROWS
    nch = DROWS // DCH
    gsems = (gsem0, gsem1)
    ssems = (ssem0, ssem1)

    pltpu.sync_copy(tok_hbm.at[pl.ds(base, DCH)], idx_v.at[0])
    pltpu.async_copy(h_hbm.at[idx_v.at[0]], rows_v.at[0], gsems[0])

    def outer(c, carry):
        for b in range(2):
            cc = 2 * c + b
            nb = 1 - b

            @pl.when(cc + 1 < nch)
            def _():
                b1 = base + (cc + 1) * DCH

                @pl.when(cc >= 1)
                def _():
                    pltpu.make_async_copy(
                        rows_v.at[nb], out_hbm.at[pl.ds(b1, DCH)],
                        ssems[nb]).wait()

                pltpu.sync_copy(tok_hbm.at[pl.ds(b1, DCH)], idx_v.at[nb])
                pltpu.async_copy(h_hbm.at[idx_v.at[nb]], rows_v.at[nb],
                                 gsems[nb])

            b0 = base + cc * DCH
            pltpu.make_async_copy(h_hbm.at[idx_v.at[b]], rows_v.at[b],
                                  gsems[b]).wait()
            pltpu.async_copy(rows_v.at[b], out_hbm.at[pl.ds(b0, DCH)],
                             ssems[b])
        return carry

    lax.fori_loop(0, nch // 2, outer, 0)
    for b in range(2):
        pltpu.make_async_copy(rows_v.at[b], out_hbm.at[pl.ds(base, DCH)],
                              ssems[b]).wait()


def _dispatch(h2, stok):
    f = functools.partial(
        pl.kernel,
        mesh=plsc.VectorSubcoreMesh(core_axis_name="c", subcore_axis_name="s"),
        out_type=jax.ShapeDtypeStruct((NPAD, D // 2), jnp.int32),
        scratch_types=[
            pltpu.VMEM((2, DCH), jnp.int32),
            pltpu.VMEM((2, DCH, D // 2), jnp.int32),
            pltpu.SemaphoreType.DMA,
            pltpu.SemaphoreType.DMA,
            pltpu.SemaphoreType.DMA,
            pltpu.SemaphoreType.DMA,
        ],
    )(_disp_body)
    return f(h2, stok)


def _gmm_body(te_ref, h_ref, sw_ref, wg_ref, wu_ref, wd_ref, y_ref):
    h = h_ref[...]
    g = jnp.dot(h, wg_ref[0], preferred_element_type=jnp.float32)
    u = jnp.dot(h, wu_ref[0], preferred_element_type=jnp.float32)
    z = (g * jax.nn.sigmoid(g) * u).astype(jnp.bfloat16)
    y = jnp.dot(z, wd_ref[0], preferred_element_type=jnp.float32)
    y_ref[...] = y * sw_ref[...]


def _gmm(texp, h_sorted, sw, wg, wu, wd):
    return pl.pallas_call(
        _gmm_body,
        grid_spec=pltpu.PrefetchScalarGridSpec(
            num_scalar_prefetch=1,
            grid=(NT,),
            in_specs=[
                pl.BlockSpec((TM, D), lambda j, te: (j, 0)),
                pl.BlockSpec((TM, 1), lambda j, te: (j, 0)),
                pl.BlockSpec((1, D, F), lambda j, te: (te[j], 0, 0)),
                pl.BlockSpec((1, D, F), lambda j, te: (te[j], 0, 0)),
                pl.BlockSpec((1, F, D), lambda j, te: (te[j], 0, 0)),
            ],
            out_specs=pl.BlockSpec((TM, D), lambda j, te: (j, 0)),
        ),
        out_shape=jax.ShapeDtypeStruct((NPAD, D), jnp.float32),
    )(texp, h_sorted, sw, wg, wu, wd)


CTOK = S // NW
CCH_T = 8


def _comb_body(x_hbm, y_hbm, pos_hbm, out_hbm, idx_v, rows_v, x_v, o_v, sem):
    wid = lax.axis_index("s") * SC_NC + lax.axis_index("c")
    base = wid * CTOK

    def chunk(c, carry):
        t0 = base + c * CCH_T
        pltpu.sync_copy(pos_hbm.at[pl.ds(t0, CCH_T)], idx_v.at[pl.ds(0, CCH_T)])
        pltpu.sync_copy(pos_hbm.at[pl.ds(S + t0, CCH_T)],
                        idx_v.at[pl.ds(CCH_T, CCH_T)])
        pltpu.async_copy(y_hbm.at[idx_v], rows_v, sem).wait()
        pltpu.sync_copy(x_hbm.at[pl.ds(t0, CCH_T)], x_v)

        def col(ci, carry2):
            sl = pl.ds(ci * 16, 16)
            for ti in range(CCH_T):
                o_v[ti, sl] = (x_v[ti, sl] + rows_v[ti, sl]
                               + rows_v[CCH_T + ti, sl])
            return carry2

        lax.fori_loop(0, D // 16, col, 0)
        pltpu.sync_copy(o_v, out_hbm.at[pl.ds(t0, CCH_T)])
        return carry

    lax.fori_loop(0, CTOK // CCH_T, chunk, 0)


def _combine(x2, y, pos):
    f = functools.partial(
        pl.kernel,
        mesh=plsc.VectorSubcoreMesh(core_axis_name="c", subcore_axis_name="s"),
        out_type=jax.ShapeDtypeStruct((S, D), jnp.float32),
        scratch_types=[
            pltpu.VMEM((2 * CCH_T,), jnp.int32),
            pltpu.VMEM((2 * CCH_T, D), jnp.float32),
            pltpu.VMEM((CCH_T, D), jnp.float32),
            pltpu.VMEM((CCH_T, D), jnp.float32),
            pltpu.SemaphoreType.DMA,
        ],
    )(_comb_body)
    return f(x2, y, pos)


def kernel(hidden_states, ln1_w, Wq, Wk, Wv, q_norm_w, k_norm_w, Wo, ln2_w,
           router_W, W_gate, W_up, W_down):
    x = hidden_states.reshape(S, D)
    w_all = jnp.concatenate([Wq, Wk, Wv], axis=1)
    qkv = _qkv(x, ln1_w, w_all, q_norm_w, k_norm_w)
    o = _attn(qkv)
    x2 = _wo(o, Wo, x)
    h2, logits = _ln2(x2, ln2_w, router_W)
    pos, w_pair, texp = _route(logits)
    stok3, sw3 = _scat(pos, w_pair)
    stok = stok3.reshape(NPAD)
    sw = sw3.reshape(NPAD, 1)
    h2i = lax.bitcast_convert_type(h2.reshape(S, D // 2, 2), jnp.int32)
    h_sorted = lax.bitcast_convert_type(
        _dispatch(h2i, stok), jnp.bfloat16).reshape(NPAD, D)
    y = _gmm(texp.reshape(NT), h_sorted, sw,
             W_gate.astype(jnp.bfloat16), W_up.astype(jnp.bfloat16),
             W_down.astype(jnp.bfloat16))
    out = _combine(x2, y, pos.reshape(K * S))
    return out.reshape(B, S, D)


# speedup vs baseline: 1.4251x; 1.4251x over previous
import functools

import jax
import jax.numpy as jnp
import numpy as np
from jax import lax
from jax.experimental import pallas as pl
from jax.experimental.pallas import tpu as pltpu
from jax.experimental.pallas import tpu_sc as plsc

B, S, D = 1, 2048, 2048
H, KVH, HD = 16, 4, 128
E, K, F = 8, 2, 768
EPS = 1e-06
THETA = 10000.0
HALF = HD // 2

TS = 256
NQKV = (H + 2 * KVH) * HD
TQ = 512
SCALE = 1.0 / float(np.sqrt(HD))


def _rms(x):
    return jax.lax.rsqrt(jnp.mean(x * x, axis=-1, keepdims=True) + EPS)


NHT = H + 2 * KVH


def _qkv_body(x_ref, ln1_ref, w_ref, wh_ref, o_ref):
    i = pl.program_id(0)
    x = x_ref[...]
    h = x * _rms(x) * ln1_ref[...]
    y = jnp.dot(h, w_ref[...], preferred_element_type=jnp.float32)
    y3 = y.reshape(TS, NHT, HD)
    yn = y3 * _rms(y3) * wh_ref[...][None]
    pos = (i * TS + lax.broadcasted_iota(jnp.int32, (TS, HALF), 0)
           ).astype(jnp.float32)
    inv = jnp.exp(lax.broadcasted_iota(jnp.int32, (TS, HALF), 1)
                  .astype(jnp.float32) * (-np.log(THETA) / HALF))
    f = pos * inv
    cos = jnp.cos(f)[:, None, :]
    sin = jnp.sin(f)[:, None, :]
    x1 = yn[..., :HALF]
    x2 = yn[..., HALF:]
    rot = jnp.concatenate([x1 * cos - x2 * sin, x2 * cos + x1 * sin], axis=-1)
    hiota = lax.broadcasted_iota(jnp.int32, (TS, NHT, HD), 1)
    o_ref[...] = jnp.where(hiota >= H + KVH, y3, rot).reshape(TS, NQKV)


def _qkv(x, ln1_w, w_all, qn, kn):
    wh = jnp.concatenate([
        jnp.broadcast_to(qn, (H, HD)),
        jnp.broadcast_to(kn, (KVH, HD)),
        jnp.ones((KVH, HD), jnp.float32),
    ], axis=0)
    return pl.pallas_call(
        _qkv_body,
        grid=(S // TS,),
        in_specs=[
            pl.BlockSpec((TS, D), lambda i: (i, 0)),
            pl.BlockSpec((1, D), lambda i: (0, 0)),
            pl.BlockSpec((D, NQKV), lambda i: (0, 0)),
            pl.BlockSpec((NHT, HD), lambda i: (0, 0)),
        ],
        out_specs=pl.BlockSpec((TS, NQKV), lambda i: (i, 0)),
        out_shape=jax.ShapeDtypeStruct((S, NQKV), jnp.float32),
    )(x, ln1_w.reshape(1, D), w_all, wh)


LOG2E = float(np.log2(np.e))


def _attn_body(ii_ref, kk_ref, q_ref, k_ref, v_ref, o_ref, m_s, l_s, a_s):
    t = pl.program_id(1)
    i = ii_ref[t]
    kt = kk_ref[t]

    @pl.when(kt == 0)
    def _():
        m_s[...] = jnp.full((TQ, 2), -1e30, jnp.float32)
        l_s[...] = jnp.zeros((TQ, 2), jnp.float32)
        a_s[...] = jnp.zeros((TQ, 2 * HD), jnp.float32)

    kb = k_ref[pl.ds(kt * TQ, TQ), :]
    vb = v_ref[pl.ds(kt * TQ, TQ), :]
    riota = lax.broadcasted_iota(jnp.int32, (TQ, TQ), 0)
    ciota = lax.broadcasted_iota(jnp.int32, (TQ, TQ), 1)
    keep = (kt < i) | (riota >= ciota)
    for a in (0, 1):
        sl = pl.ds(a * HD, HD)
        qa = q_ref[:, sl] * (SCALE * LOG2E)
        s = lax.dot_general(qa, kb, (((1,), (1,)), ((), ())),
                            preferred_element_type=jnp.float32)
        s = jnp.where(keep, s, -1e30)
        m_prev = m_s[:, a:a + 1]
        m_new = jnp.maximum(m_prev, jnp.max(s, axis=1, keepdims=True))
        alpha = jnp.exp2(m_prev - m_new)
        p = jnp.exp2(s - m_new)
        l_new = l_s[:, a:a + 1] * alpha + jnp.sum(p, axis=1, keepdims=True)
        a_new = (a_s[:, sl] * alpha
                 + lax.dot_general(p, vb, (((1,), (0,)), ((), ())),
                                   preferred_element_type=jnp.float32))
        m_s[:, a:a + 1] = m_new
        l_s[:, a:a + 1] = l_new
        a_s[:, sl] = a_new

    @pl.when(kt == i)
    def _():
        for a in (0, 1):
            sl = pl.ds(a * HD, HD)
            o_ref[:, sl] = a_s[:, sl] / l_s[:, a:a + 1]


def _attn(qkv):
    nq = S // TQ
    rep = H // KVH
    tri = [(i, k) for i in range(nq) for k in range(i + 1)]
    iidx = jnp.asarray(np.array([x[0] for x in tri], np.int32))
    kidx = jnp.asarray(np.array([x[1] for x in tri], np.int32))
    return pl.pallas_call(
        _attn_body,
        grid_spec=pltpu.PrefetchScalarGridSpec(
            num_scalar_prefetch=2,
            grid=(H // 2, len(tri)),
            in_specs=[
                pl.BlockSpec((TQ, 2 * HD), lambda h2, t, ii, kk: (ii[t], h2)),
                pl.BlockSpec((S, HD),
                             lambda h2, t, ii, kk: (0, H + (2 * h2) // rep)),
                pl.BlockSpec((S, HD),
                             lambda h2, t, ii, kk: (0, H + KVH + (2 * h2) // rep)),
            ],
            out_specs=pl.BlockSpec((TQ, 2 * HD),
                                   lambda h2, t, ii, kk: (ii[t], h2)),
            scratch_shapes=[
                pltpu.VMEM((TQ, 2), jnp.float32),
                pltpu.VMEM((TQ, 2), jnp.float32),
                pltpu.VMEM((TQ, 2 * HD), jnp.float32),
            ],
        ),
        out_shape=jax.ShapeDtypeStruct((S, H * HD), jnp.float32),
        compiler_params=pltpu.CompilerParams(
            dimension_semantics=("parallel", "arbitrary")),
    )(iidx, kidx, qkv, qkv, qkv)


TNO = 512


def _wo_body(o_ref, w_ref, r_ref, y_ref):
    y_ref[...] = r_ref[...] + jnp.dot(o_ref[...], w_ref[...],
                                      preferred_element_type=jnp.float32)


def _wo(o, wo, resid):
    return pl.pallas_call(
        _wo_body,
        grid=(S // TS, D // TNO),
        in_specs=[
            pl.BlockSpec((TS, H * HD), lambda i, j: (i, 0)),
            pl.BlockSpec((H * HD, TNO), lambda i, j: (0, j)),
            pl.BlockSpec((TS, TNO), lambda i, j: (i, j)),
        ],
        out_specs=pl.BlockSpec((TS, TNO), lambda i, j: (i, j)),
        out_shape=jax.ShapeDtypeStruct((S, D), jnp.float32),
    )(o, wo, resid)


def _ln2_body(x_ref, w_ref, rw_ref, h_ref, lg_ref):
    x = x_ref[...]
    hh = x * _rms(x) * w_ref[...]
    h_ref[...] = hh
    lg_ref[...] = jnp.dot(hh, rw_ref[...], preferred_element_type=jnp.float32)


def _ln2(x, ln2_w, router_W):
    return pl.pallas_call(
        _ln2_body,
        grid=(S // TS,),
        in_specs=[
            pl.BlockSpec((TS, D), lambda i: (i, 0)),
            pl.BlockSpec((1, D), lambda i: (0, 0)),
            pl.BlockSpec((D, E), lambda i: (0, 0)),
        ],
        out_specs=[
            pl.BlockSpec((TS, D), lambda i: (i, 0)),
            pl.BlockSpec((TS, E), lambda i: (i, 0)),
        ],
        out_shape=[
            jax.ShapeDtypeStruct((S, D), jnp.float32),
            jax.ShapeDtypeStruct((S, E), jnp.float32),
        ],
    )(x, ln2_w.reshape(1, D), router_W)


TM = 256
NT = (K * S) // TM + E
NPAD = NT * TM
CCH = 128
NCH = (K * S) // CCH


def _route_body(lg_ref, pos_ref, w_ref, texp_ref):
    lg = lg_ref[...]
    m = jnp.max(lg, axis=1, keepdims=True)
    p = jnp.exp(lg - m)
    p = p / jnp.sum(p, axis=1, keepdims=True)
    iota = lax.broadcasted_iota(jnp.int32, (S, E), 1)
    m1 = jnp.max(p, axis=1, keepdims=True)
    i1 = jnp.min(jnp.where(p == m1, iota, E), axis=1, keepdims=True)
    p2 = jnp.where(iota == i1, -1.0, p)
    m2 = jnp.max(p2, axis=1, keepdims=True)
    i2 = jnp.min(jnp.where(p2 == m2, iota, E), axis=1, keepdims=True)
    denom = m1 + m2
    w_ref[...] = jnp.concatenate([m1 / denom, m2 / denom], axis=0)
    idx_all = jnp.concatenate([i1, i2], axis=0)
    M = (lax.broadcasted_iota(jnp.int32, (K * S, E), 1) == idx_all
         ).astype(jnp.float32)
    M3 = M.reshape(NCH, CCH, E)
    tri = (lax.broadcasted_iota(jnp.int32, (CCH, CCH), 1)
           < lax.broadcasted_iota(jnp.int32, (CCH, CCH), 0)).astype(jnp.float32)
    trib = jnp.broadcast_to(tri, (NCH, CCH, CCH))
    pre = lax.dot_general(trib, M3, (((2,), (1,)), ((0,), (0,))),
                          preferred_element_type=jnp.float32)
    tot = jnp.sum(M3, axis=1)
    tri2 = (lax.broadcasted_iota(jnp.int32, (NCH, NCH), 1)
            < lax.broadcasted_iota(jnp.int32, (NCH, NCH), 0)).astype(jnp.float32)
    coff = jnp.dot(tri2, tot, preferred_element_type=jnp.float32)
    rank = (pre + coff[:, None, :]).reshape(K * S, E)
    counts = jnp.sum(M, axis=0, keepdims=True)
    pc = jnp.floor((counts + (TM - 1)) * (1.0 / TM)) * TM
    triu = (lax.broadcasted_iota(jnp.int32, (E, E), 0)
            < lax.broadcasted_iota(jnp.int32, (E, E), 1)).astype(jnp.float32)
    pad_off = jnp.dot(pc, triu, preferred_element_type=jnp.float32)
    posf = jnp.sum((rank + pad_off) * M, axis=1, keepdims=True)
    pos_ref[...] = posf.astype(jnp.int32)
    pad_end = pad_off + pc
    jtf = (lax.broadcasted_iota(jnp.int32, (NT, E), 0) * TM).astype(jnp.float32)
    texp = jnp.sum((pad_end <= jtf).astype(jnp.int32), axis=1, keepdims=True)
    texp_ref[...] = jnp.minimum(texp, E - 1)


def _route(logits):
    return pl.pallas_call(
        _route_body,
        out_shape=[
            jax.ShapeDtypeStruct((K * S, 1), jnp.int32),
            jax.ShapeDtypeStruct((K * S, 1), jnp.float32),
            jax.ShapeDtypeStruct((NT, 1), jnp.int32),
        ],
    )(logits)


SCH = 256


def _scat_body(pos_ref, w_ref, stok_ref, sw_ref):
    jcols = pl.program_id(0) * SCH + lax.broadcasted_iota(
        jnp.int32, (K * S, SCH), 1)
    cmp = pos_ref[...] == jcols
    it = lax.broadcasted_iota(jnp.int32, (K * S, 1), 0)
    tok = jnp.where(it >= S, it - S, it)
    stok = jnp.sum(jnp.where(cmp, tok, 0), axis=0, keepdims=True)
    sw = jnp.sum(jnp.where(cmp, w_ref[...], 0.0), axis=0, keepdims=True)
    stok_ref[...] = stok.reshape(1, 1, SCH)
    sw_ref[...] = sw.reshape(1, 1, SCH)


def _scat(pos, w):
    return pl.pallas_call(
        _scat_body,
        grid=(NPAD // SCH,),
        in_specs=[
            pl.BlockSpec((K * S, 1), lambda j: (0, 0)),
            pl.BlockSpec((K * S, 1), lambda j: (0, 0)),
        ],
        out_specs=[
            pl.BlockSpec((1, 1, SCH), lambda j: (j, 0, 0)),
            pl.BlockSpec((1, 1, SCH), lambda j: (j, 0, 0)),
        ],
        out_shape=[
            jax.ShapeDtypeStruct((NPAD // SCH, 1, SCH), jnp.int32),
            jax.ShapeDtypeStruct((NPAD // SCH, 1, SCH), jnp.float32),
        ],
    )(pos, w)


SC_NC, SC_NS = 2, 16
NW = SC_NC * SC_NS
DROWS = NPAD // NW
DCH = 24


def _disp_body(h_hbm, tok_hbm, out_hbm, idx_v, rows_v,
               gsem0, gsem1, ssem0, ssem1):
    wid = lax.axis_index("s") * SC_NC + lax.axis_index("c")
    base = wid * DROWS
    nch = DROWS // DCH
    gsems = (gsem0, gsem1)
    ssems = (ssem0, ssem1)

    pltpu.sync_copy(tok_hbm.at[pl.ds(base, DCH)], idx_v.at[0])
    pltpu.async_copy(h_hbm.at[idx_v.at[0]], rows_v.at[0], gsems[0])

    def outer(c, carry):
        for b in range(2):
            cc = 2 * c + b
            nb = 1 - b

            @pl.when(cc + 1 < nch)
            def _():
                b1 = base + (cc + 1) * DCH

                @pl.when(cc >= 1)
                def _():
                    pltpu.make_async_copy(
                        rows_v.at[nb], out_hbm.at[pl.ds(b1, DCH)],
                        ssems[nb]).wait()

                pltpu.sync_copy(tok_hbm.at[pl.ds(b1, DCH)], idx_v.at[nb])
                pltpu.async_copy(h_hbm.at[idx_v.at[nb]], rows_v.at[nb],
                                 gsems[nb])

            b0 = base + cc * DCH
            pltpu.make_async_copy(h_hbm.at[idx_v.at[b]], rows_v.at[b],
                                  gsems[b]).wait()
            pltpu.async_copy(rows_v.at[b], out_hbm.at[pl.ds(b0, DCH)],
                             ssems[b])
        return carry

    lax.fori_loop(0, nch // 2, outer, 0)
    for b in range(2):
        pltpu.make_async_copy(rows_v.at[b], out_hbm.at[pl.ds(base, DCH)],
                              ssems[b]).wait()


def _dispatch(h2, stok):
    f = functools.partial(
        pl.kernel,
        mesh=plsc.VectorSubcoreMesh(core_axis_name="c", subcore_axis_name="s"),
        out_type=jax.ShapeDtypeStruct((NPAD, D), jnp.float32),
        scratch_types=[
            pltpu.VMEM((2, DCH), jnp.int32),
            pltpu.VMEM((2, DCH, D), jnp.float32),
            pltpu.SemaphoreType.DMA,
            pltpu.SemaphoreType.DMA,
            pltpu.SemaphoreType.DMA,
            pltpu.SemaphoreType.DMA,
        ],
    )(_disp_body)
    return f(h2, stok)


def _gmm_body(te_ref, h_ref, sw_ref, wg_ref, wu_ref, wd_ref, y_ref):
    h = h_ref[...].astype(jnp.bfloat16)
    g = jnp.dot(h, wg_ref[0].astype(jnp.bfloat16),
                preferred_element_type=jnp.float32)
    u = jnp.dot(h, wu_ref[0].astype(jnp.bfloat16),
                preferred_element_type=jnp.float32)
    z = (g * jax.nn.sigmoid(g) * u).astype(jnp.bfloat16)
    y = jnp.dot(z, wd_ref[0].astype(jnp.bfloat16),
                preferred_element_type=jnp.float32)
    y_ref[...] = y * sw_ref[...]


def _gmm(texp, h_sorted, sw, wg, wu, wd):
    return pl.pallas_call(
        _gmm_body,
        grid_spec=pltpu.PrefetchScalarGridSpec(
            num_scalar_prefetch=1,
            grid=(NT,),
            in_specs=[
                pl.BlockSpec((TM, D), lambda j, te: (j, 0)),
                pl.BlockSpec((TM, 1), lambda j, te: (j, 0)),
                pl.BlockSpec((1, D, F), lambda j, te: (te[j], 0, 0)),
                pl.BlockSpec((1, D, F), lambda j, te: (te[j], 0, 0)),
                pl.BlockSpec((1, F, D), lambda j, te: (te[j], 0, 0)),
            ],
            out_specs=pl.BlockSpec((TM, D), lambda j, te: (j, 0)),
        ),
        out_shape=jax.ShapeDtypeStruct((NPAD, D), jnp.float32),
    )(texp, h_sorted, sw, wg, wu, wd)


CTOK = S // NW
CCH_T = 8


def _comb_body(x_hbm, y_hbm, pos_hbm, out_hbm, idx_v, rows_v, x_v, o_v, sem):
    wid = lax.axis_index("s") * SC_NC + lax.axis_index("c")
    base = wid * CTOK

    def chunk(c, carry):
        t0 = base + c * CCH_T
        pltpu.sync_copy(pos_hbm.at[pl.ds(t0, CCH_T)], idx_v.at[pl.ds(0, CCH_T)])
        pltpu.sync_copy(pos_hbm.at[pl.ds(S + t0, CCH_T)],
                        idx_v.at[pl.ds(CCH_T, CCH_T)])
        pltpu.async_copy(y_hbm.at[idx_v], rows_v, sem).wait()
        pltpu.sync_copy(x_hbm.at[pl.ds(t0, CCH_T)], x_v)

        def col(ci, carry2):
            sl = pl.ds(ci * 16, 16)
            for ti in range(CCH_T):
                o_v[ti, sl] = (x_v[ti, sl] + rows_v[ti, sl]
                               + rows_v[CCH_T + ti, sl])
            return carry2

        lax.fori_loop(0, D // 16, col, 0)
        pltpu.sync_copy(o_v, out_hbm.at[pl.ds(t0, CCH_T)])
        return carry

    lax.fori_loop(0, CTOK // CCH_T, chunk, 0)


def _combine(x2, y, pos):
    f = functools.partial(
        pl.kernel,
        mesh=plsc.VectorSubcoreMesh(core_axis_name="c", subcore_axis_name="s"),
        out_type=jax.ShapeDtypeStruct((S, D), jnp.float32),
        scratch_types=[
            pltpu.VMEM((2 * CCH_T,), jnp.int32),
            pltpu.VMEM((2 * CCH_T, D), jnp.float32),
            pltpu.VMEM((CCH_T, D), jnp.float32),
            pltpu.VMEM((CCH_T, D), jnp.float32),
            pltpu.SemaphoreType.DMA,
        ],
    )(_comb_body)
    return f(x2, y, pos)


def kernel(hidden_states, ln1_w, Wq, Wk, Wv, q_norm_w, k_norm_w, Wo, ln2_w,
           router_W, W_gate, W_up, W_down):
    x = hidden_states.reshape(S, D)
    w_all = jnp.concatenate([Wq, Wk, Wv], axis=1)
    qkv = _qkv(x, ln1_w, w_all, q_norm_w, k_norm_w)
    o = _attn(qkv)
    x2 = _wo(o, Wo, x)
    h2, logits = _ln2(x2, ln2_w, router_W)
    pos, w_pair, texp = _route(logits)
    stok3, sw3 = _scat(pos, w_pair)
    stok = stok3.reshape(NPAD)
    sw = sw3.reshape(NPAD, 1)
    h_sorted = _dispatch(h2, stok)
    y = _gmm(texp.reshape(NT), h_sorted, sw, W_gate, W_up, W_down)
    out = _combine(x2, y, pos.reshape(K * S))
    return out.reshape(B, S, D)


# speedup vs baseline: 1.6714x; 1.1728x over previous
import functools

import jax
import jax.numpy as jnp
import numpy as np
from jax import lax
from jax.experimental import pallas as pl
from jax.experimental.pallas import tpu as pltpu
from jax.experimental.pallas import tpu_sc as plsc

B, S, D = 1, 2048, 2048
H, KVH, HD = 16, 4, 128
E, K, F = 8, 2, 768
EPS = 1e-06
THETA = 10000.0
HALF = HD // 2

TS = 256
NQKV = (H + 2 * KVH) * HD
TQ = 512
SCALE = 1.0 / float(np.sqrt(HD))


def _rms(x):
    return jax.lax.rsqrt(jnp.mean(x * x, axis=-1, keepdims=True) + EPS)


NHT = H + 2 * KVH


def _qkv_body(x_ref, ln1_ref, w_ref, wh_ref, o_ref):
    i = pl.program_id(0)
    x = x_ref[...]
    h = x * _rms(x) * ln1_ref[...]
    y = jnp.dot(h, w_ref[...], preferred_element_type=jnp.float32)
    y3 = y.reshape(TS, NHT, HD)
    yn = y3 * _rms(y3) * wh_ref[...][None]
    pos = (i * TS + lax.broadcasted_iota(jnp.int32, (TS, HALF), 0)
           ).astype(jnp.float32)
    inv = jnp.exp(lax.broadcasted_iota(jnp.int32, (TS, HALF), 1)
                  .astype(jnp.float32) * (-np.log(THETA) / HALF))
    f = pos * inv
    cos = jnp.cos(f)[:, None, :]
    sin = jnp.sin(f)[:, None, :]
    x1 = yn[..., :HALF]
    x2 = yn[..., HALF:]
    rot = jnp.concatenate([x1 * cos - x2 * sin, x2 * cos + x1 * sin], axis=-1)
    hiota = lax.broadcasted_iota(jnp.int32, (TS, NHT, HD), 1)
    o_ref[...] = jnp.where(hiota >= H + KVH, y3, rot).reshape(TS, NQKV)


def _qkv(x, ln1_w, w_all, qn, kn):
    wh = jnp.concatenate([
        jnp.broadcast_to(qn, (H, HD)),
        jnp.broadcast_to(kn, (KVH, HD)),
        jnp.ones((KVH, HD), jnp.float32),
    ], axis=0)
    return pl.pallas_call(
        _qkv_body,
        grid=(S // TS,),
        in_specs=[
            pl.BlockSpec((TS, D), lambda i: (i, 0)),
            pl.BlockSpec((1, D), lambda i: (0, 0)),
            pl.BlockSpec((D, NQKV), lambda i: (0, 0)),
            pl.BlockSpec((NHT, HD), lambda i: (0, 0)),
        ],
        out_specs=pl.BlockSpec((TS, NQKV), lambda i: (i, 0)),
        out_shape=jax.ShapeDtypeStruct((S, NQKV), jnp.float32),
    )(x, ln1_w.reshape(1, D), w_all, wh)


LOG2E = float(np.log2(np.e))


def _attn_body(ii_ref, kk_ref, q_ref, k_ref, v_ref, o_ref, m_s, l_s, a_s):
    t = pl.program_id(1)
    i = ii_ref[t]
    kt = kk_ref[t]

    @pl.when(kt == 0)
    def _():
        m_s[...] = jnp.full((TQ, 2), -1e30, jnp.float32)
        l_s[...] = jnp.zeros((TQ, 2), jnp.float32)
        a_s[...] = jnp.zeros((TQ, 2 * HD), jnp.float32)

    kb = k_ref[pl.ds(kt * TQ, TQ), :]
    vb = v_ref[pl.ds(kt * TQ, TQ), :]
    riota = lax.broadcasted_iota(jnp.int32, (TQ, TQ), 0)
    ciota = lax.broadcasted_iota(jnp.int32, (TQ, TQ), 1)
    keep = (kt < i) | (riota >= ciota)
    for a in (0, 1):
        sl = pl.ds(a * HD, HD)
        qa = q_ref[:, sl] * (SCALE * LOG2E)
        s = lax.dot_general(qa, kb, (((1,), (1,)), ((), ())),
                            preferred_element_type=jnp.float32)
        s = jnp.where(keep, s, -1e30)
        m_prev = m_s[:, a:a + 1]
        m_new = jnp.maximum(m_prev, jnp.max(s, axis=1, keepdims=True))
        alpha = jnp.exp2(m_prev - m_new)
        p = jnp.exp2(s - m_new)
        l_new = l_s[:, a:a + 1] * alpha + jnp.sum(p, axis=1, keepdims=True)
        a_new = (a_s[:, sl] * alpha
                 + lax.dot_general(p, vb, (((1,), (0,)), ((), ())),
                                   preferred_element_type=jnp.float32))
        m_s[:, a:a + 1] = m_new
        l_s[:, a:a + 1] = l_new
        a_s[:, sl] = a_new

    @pl.when(kt == i)
    def _():
        for a in (0, 1):
            sl = pl.ds(a * HD, HD)
            o_ref[:, sl] = a_s[:, sl] / l_s[:, a:a + 1]


def _attn(qkv):
    nq = S // TQ
    rep = H // KVH
    tri = [(i, k) for i in range(nq) for k in range(i + 1)]
    iidx = jnp.asarray(np.array([x[0] for x in tri], np.int32))
    kidx = jnp.asarray(np.array([x[1] for x in tri], np.int32))
    return pl.pallas_call(
        _attn_body,
        grid_spec=pltpu.PrefetchScalarGridSpec(
            num_scalar_prefetch=2,
            grid=(H // 2, len(tri)),
            in_specs=[
                pl.BlockSpec((TQ, 2 * HD), lambda h2, t, ii, kk: (ii[t], h2)),
                pl.BlockSpec((S, HD),
                             lambda h2, t, ii, kk: (0, H + (2 * h2) // rep)),
                pl.BlockSpec((S, HD),
                             lambda h2, t, ii, kk: (0, H + KVH + (2 * h2) // rep)),
            ],
            out_specs=pl.BlockSpec((TQ, 2 * HD),
                                   lambda h2, t, ii, kk: (ii[t], h2)),
            scratch_shapes=[
                pltpu.VMEM((TQ, 2), jnp.float32),
                pltpu.VMEM((TQ, 2), jnp.float32),
                pltpu.VMEM((TQ, 2 * HD), jnp.float32),
            ],
        ),
        out_shape=jax.ShapeDtypeStruct((S, H * HD), jnp.float32),
        compiler_params=pltpu.CompilerParams(
            dimension_semantics=("parallel", "arbitrary")),
    )(iidx, kidx, qkv, qkv, qkv)


def _wo_body(o_ref, w_ref, r_ref, lw_ref, rw_ref, x2_ref, h_ref, lg_ref):
    x2 = r_ref[...] + jnp.dot(o_ref[...], w_ref[...],
                              preferred_element_type=jnp.float32)
    x2_ref[...] = x2
    hh = x2 * _rms(x2) * lw_ref[...]
    h_ref[...] = hh
    lg_ref[...] = jnp.dot(hh, rw_ref[...], preferred_element_type=jnp.float32)


def _wo(o, wo, resid, ln2_w, router_W):
    return pl.pallas_call(
        _wo_body,
        grid=(S // TS,),
        in_specs=[
            pl.BlockSpec((TS, H * HD), lambda i: (i, 0)),
            pl.BlockSpec((H * HD, D), lambda i: (0, 0)),
            pl.BlockSpec((TS, D), lambda i: (i, 0)),
            pl.BlockSpec((1, D), lambda i: (0, 0)),
            pl.BlockSpec((D, E), lambda i: (0, 0)),
        ],
        out_specs=[
            pl.BlockSpec((TS, D), lambda i: (i, 0)),
            pl.BlockSpec((TS, D), lambda i: (i, 0)),
            pl.BlockSpec((TS, E), lambda i: (i, 0)),
        ],
        out_shape=[
            jax.ShapeDtypeStruct((S, D), jnp.float32),
            jax.ShapeDtypeStruct((S, D), jnp.float32),
            jax.ShapeDtypeStruct((S, E), jnp.float32),
        ],
    )(o, wo, resid, ln2_w.reshape(1, D), router_W)


TM = 128
NT = (K * S) // TM + E
NPAD = NT * TM
CCH = 128
NCH = (K * S) // CCH


def _route_body(lg_ref, pos_ref, w_ref, texp_ref):
    lg = lg_ref[...]
    m = jnp.max(lg, axis=1, keepdims=True)
    p = jnp.exp(lg - m)
    p = p / jnp.sum(p, axis=1, keepdims=True)
    iota = lax.broadcasted_iota(jnp.int32, (S, E), 1)
    m1 = jnp.max(p, axis=1, keepdims=True)
    i1 = jnp.min(jnp.where(p == m1, iota, E), axis=1, keepdims=True)
    p2 = jnp.where(iota == i1, -1.0, p)
    m2 = jnp.max(p2, axis=1, keepdims=True)
    i2 = jnp.min(jnp.where(p2 == m2, iota, E), axis=1, keepdims=True)
    denom = m1 + m2
    w_ref[...] = jnp.concatenate([m1 / denom, m2 / denom], axis=0)
    idx_all = jnp.concatenate([i1, i2], axis=0)
    M = (lax.broadcasted_iota(jnp.int32, (K * S, E), 1) == idx_all
         ).astype(jnp.float32)
    M3 = M.reshape(NCH, CCH, E)
    tri = (lax.broadcasted_iota(jnp.int32, (CCH, CCH), 1)
           < lax.broadcasted_iota(jnp.int32, (CCH, CCH), 0)).astype(jnp.float32)
    trib = jnp.broadcast_to(tri, (NCH, CCH, CCH))
    pre = lax.dot_general(trib, M3, (((2,), (1,)), ((0,), (0,))),
                          preferred_element_type=jnp.float32)
    tot = jnp.sum(M3, axis=1)
    tri2 = (lax.broadcasted_iota(jnp.int32, (NCH, NCH), 1)
            < lax.broadcasted_iota(jnp.int32, (NCH, NCH), 0)).astype(jnp.float32)
    coff = jnp.dot(tri2, tot, preferred_element_type=jnp.float32)
    rank = (pre + coff[:, None, :]).reshape(K * S, E)
    counts = jnp.sum(M, axis=0, keepdims=True)
    pc = jnp.floor((counts + (TM - 1)) * (1.0 / TM)) * TM
    triu = (lax.broadcasted_iota(jnp.int32, (E, E), 0)
            < lax.broadcasted_iota(jnp.int32, (E, E), 1)).astype(jnp.float32)
    pad_off = jnp.dot(pc, triu, preferred_element_type=jnp.float32)
    posf = jnp.sum((rank + pad_off) * M, axis=1, keepdims=True)
    pos_ref[...] = posf.astype(jnp.int32)
    pad_end = pad_off + pc
    jtf = (lax.broadcasted_iota(jnp.int32, (NT, E), 0) * TM).astype(jnp.float32)
    texp = jnp.sum((pad_end <= jtf).astype(jnp.int32), axis=1, keepdims=True)
    texp_ref[...] = jnp.minimum(texp, E - 1)


def _route(logits):
    return pl.pallas_call(
        _route_body,
        out_shape=[
            jax.ShapeDtypeStruct((K * S, 1), jnp.int32),
            jax.ShapeDtypeStruct((K * S, 1), jnp.float32),
            jax.ShapeDtypeStruct((NT, 1), jnp.int32),
        ],
    )(logits)


SCH = 256


def _scat_body(pos_ref, w_ref, stok_ref, sw_ref):
    jcols = pl.program_id(0) * SCH + lax.broadcasted_iota(
        jnp.int32, (K * S, SCH), 1)
    cmp = pos_ref[...] == jcols
    it = lax.broadcasted_iota(jnp.int32, (K * S, 1), 0)
    tok = jnp.where(it >= S, it - S, it)
    stok = jnp.sum(jnp.where(cmp, tok, 0), axis=0, keepdims=True)
    sw = jnp.sum(jnp.where(cmp, w_ref[...], 0.0), axis=0, keepdims=True)
    stok_ref[...] = stok.reshape(1, 1, SCH)
    sw_ref[...] = sw.reshape(1, 1, SCH)


def _scat(pos, w):
    return pl.pallas_call(
        _scat_body,
        grid=(NPAD // SCH,),
        in_specs=[
            pl.BlockSpec((K * S, 1), lambda j: (0, 0)),
            pl.BlockSpec((K * S, 1), lambda j: (0, 0)),
        ],
        out_specs=[
            pl.BlockSpec((1, 1, SCH), lambda j: (j, 0, 0)),
            pl.BlockSpec((1, 1, SCH), lambda j: (j, 0, 0)),
        ],
        out_shape=[
            jax.ShapeDtypeStruct((NPAD // SCH, 1, SCH), jnp.int32),
            jax.ShapeDtypeStruct((NPAD // SCH, 1, SCH), jnp.float32),
        ],
    )(pos, w)


SC_NC, SC_NS = 2, 16
NW = SC_NC * SC_NS
DROWS = NPAD // NW
DCH = 16


def _disp_body(h_hbm, tok_hbm, out_hbm, idx_v, rows_v,
               gsem0, gsem1, ssem0, ssem1):
    wid = lax.axis_index("s") * SC_NC + lax.axis_index("c")
    base = wid * DROWS
    nch = DROWS // DCH
    gsems = (gsem0, gsem1)
    ssems = (ssem0, ssem1)

    pltpu.sync_copy(tok_hbm.at[pl.ds(base, DCH)], idx_v.at[0])
    pltpu.async_copy(h_hbm.at[idx_v.at[0]], rows_v.at[0], gsems[0])

    def outer(c, carry):
        for b in range(2):
            cc = 2 * c + b
            nb = 1 - b

            @pl.when(cc + 1 < nch)
            def _():
                b1 = base + (cc + 1) * DCH

                @pl.when(cc >= 1)
                def _():
                    pltpu.make_async_copy(
                        rows_v.at[nb], out_hbm.at[pl.ds(b1, DCH)],
                        ssems[nb]).wait()

                pltpu.sync_copy(tok_hbm.at[pl.ds(b1, DCH)], idx_v.at[nb])
                pltpu.async_copy(h_hbm.at[idx_v.at[nb]], rows_v.at[nb],
                                 gsems[nb])

            b0 = base + cc * DCH
            pltpu.make_async_copy(h_hbm.at[idx_v.at[b]], rows_v.at[b],
                                  gsems[b]).wait()
            pltpu.async_copy(rows_v.at[b], out_hbm.at[pl.ds(b0, DCH)],
                             ssems[b])
        return carry

    lax.fori_loop(0, nch // 2, outer, 0)
    for b in range(2):
        pltpu.make_async_copy(rows_v.at[b], out_hbm.at[pl.ds(base, DCH)],
                              ssems[b]).wait()


def _dispatch(h2, stok):
    f = functools.partial(
        pl.kernel,
        mesh=plsc.VectorSubcoreMesh(core_axis_name="c", subcore_axis_name="s"),
        out_type=jax.ShapeDtypeStruct((NPAD, D), jnp.float32),
        scratch_types=[
            pltpu.VMEM((2, DCH), jnp.int32),
            pltpu.VMEM((2, DCH, D), jnp.float32),
            pltpu.SemaphoreType.DMA,
            pltpu.SemaphoreType.DMA,
            pltpu.SemaphoreType.DMA,
            pltpu.SemaphoreType.DMA,
        ],
    )(_disp_body)
    return f(h2, stok)


def _gmm_body(te_ref, h_ref, sw_ref, wg_ref, wu_ref, wd_ref, y_ref):
    h = h_ref[...].astype(jnp.bfloat16)
    g = jnp.dot(h, wg_ref[0].astype(jnp.bfloat16),
                preferred_element_type=jnp.float32)
    u = jnp.dot(h, wu_ref[0].astype(jnp.bfloat16),
                preferred_element_type=jnp.float32)
    z = (g * jax.nn.sigmoid(g) * u).astype(jnp.bfloat16)
    y = jnp.dot(z, wd_ref[0].astype(jnp.bfloat16),
                preferred_element_type=jnp.float32)
    y_ref[...] = y * sw_ref[...]


def _gmm(texp, h_sorted, sw, wg, wu, wd):
    return pl.pallas_call(
        _gmm_body,
        grid_spec=pltpu.PrefetchScalarGridSpec(
            num_scalar_prefetch=1,
            grid=(NT,),
            in_specs=[
                pl.BlockSpec((TM, D), lambda j, te: (j, 0)),
                pl.BlockSpec((TM, 1), lambda j, te: (j, 0)),
                pl.BlockSpec((1, D, F), lambda j, te: (te[j], 0, 0)),
                pl.BlockSpec((1, D, F), lambda j, te: (te[j], 0, 0)),
                pl.BlockSpec((1, F, D), lambda j, te: (te[j], 0, 0)),
            ],
            out_specs=pl.BlockSpec((TM, D), lambda j, te: (j, 0)),
        ),
        out_shape=jax.ShapeDtypeStruct((NPAD, D), jnp.float32),
    )(texp, h_sorted, sw, wg, wu, wd)


CTOK = S // NW
CCH_T = 8


def _comb_body(x_hbm, y_hbm, pos_hbm, out_hbm, idx_v, rows_v, x_v, o_v, sem):
    wid = lax.axis_index("s") * SC_NC + lax.axis_index("c")
    base = wid * CTOK

    def chunk(c, carry):
        t0 = base + c * CCH_T
        pltpu.sync_copy(pos_hbm.at[pl.ds(t0, CCH_T)], idx_v.at[pl.ds(0, CCH_T)])
        pltpu.sync_copy(pos_hbm.at[pl.ds(S + t0, CCH_T)],
                        idx_v.at[pl.ds(CCH_T, CCH_T)])
        pltpu.async_copy(y_hbm.at[idx_v], rows_v, sem).wait()
        pltpu.sync_copy(x_hbm.at[pl.ds(t0, CCH_T)], x_v)

        def col(ci, carry2):
            sl = pl.ds(ci * 16, 16)
            for ti in range(CCH_T):
                o_v[ti, sl] = (x_v[ti, sl] + rows_v[ti, sl]
                               + rows_v[CCH_T + ti, sl])
            return carry2

        lax.fori_loop(0, D // 16, col, 0)
        pltpu.sync_copy(o_v, out_hbm.at[pl.ds(t0, CCH_T)])
        return carry

    lax.fori_loop(0, CTOK // CCH_T, chunk, 0)


def _combine(x2, y, pos):
    f = functools.partial(
        pl.kernel,
        mesh=plsc.VectorSubcoreMesh(core_axis_name="c", subcore_axis_name="s"),
        out_type=jax.ShapeDtypeStruct((S, D), jnp.float32),
        scratch_types=[
            pltpu.VMEM((2 * CCH_T,), jnp.int32),
            pltpu.VMEM((2 * CCH_T, D), jnp.float32),
            pltpu.VMEM((CCH_T, D), jnp.float32),
            pltpu.VMEM((CCH_T, D), jnp.float32),
            pltpu.SemaphoreType.DMA,
        ],
    )(_comb_body)
    return f(x2, y, pos)


def kernel(hidden_states, ln1_w, Wq, Wk, Wv, q_norm_w, k_norm_w, Wo, ln2_w,
           router_W, W_gate, W_up, W_down):
    x = hidden_states.reshape(S, D)
    w_all = jnp.concatenate([Wq, Wk, Wv], axis=1)
    qkv = _qkv(x, ln1_w, w_all, q_norm_w, k_norm_w)
    o = _attn(qkv)
    x2, h2, logits = _wo(o, Wo, x, ln2_w, router_W)
    pos, w_pair, texp = _route(logits)
    stok3, sw3 = _scat(pos, w_pair)
    stok = stok3.reshape(NPAD)
    sw = sw3.reshape(NPAD, 1)
    h_sorted = _dispatch(h2, stok)
    y = _gmm(texp.reshape(NT), h_sorted, sw, W_gate, W_up, W_down)
    out = _combine(x2, y, pos.reshape(K * S))
    return out.reshape(B, S, D)
